# Initial kernel scaffold; baseline (speedup 1.0000x reference)
import jax
import jax.numpy as jnp
from jax.experimental import pallas as pl


def kernel(edge_features, edge_index, edge_type, scales, shifts):
    raise NotImplementedError("write your pallas kernel here")


# speedup vs baseline: 9.3145x; 9.3145x over previous
import functools

import numpy as np
import jax
import jax.numpy as jnp
from jax import lax
from jax.experimental import pallas as pl
from jax.experimental.pallas import tpu as pltpu
from jax.experimental.pallas import tpu_sc as plsc

NUM_PAIRS = 4096
IRREPS_DIM = 128
NUM_IRREPS = 64
NUM_SCALAR = 32
N_EDGES = 320000

_SCALE_INDEX = np.concatenate(
    [np.arange(32), np.repeat(np.arange(32, 64), 3)]
).astype(np.int32)

_INFO = plsc.get_sparse_core_info()
_NC, _NS, _L = _INFO.num_cores, _INFO.num_subcores, _INFO.num_lanes
_NW = _NC * _NS
_PER_W = N_EDGES // _NW
_B = 200
_CHUNKS = _PER_W // _B


def _sc_body(feat_hbm, etype_hbm, scales_hbm, shifts_hbm, out_hbm,
             idx_v, x_v, s_v, h_v, sem_x, sem_s, sem_h):
    wid = lax.axis_index("s") * _NC + lax.axis_index("c")
    w_base = wid * _PER_W

    def chunk(g, carry):
        base = w_base + g * _B
        pltpu.sync_copy(etype_hbm.at[pl.ds(base, _B)], idx_v)
        cp_x = pltpu.async_copy(feat_hbm.at[pl.ds(base, _B)], x_v, sem_x)
        cp_s = pltpu.async_copy(scales_hbm.at[idx_v], s_v, sem_s)
        cp_h = pltpu.async_copy(shifts_hbm.at[idx_v], h_v, sem_h)
        cp_x.wait()
        cp_s.wait()
        cp_h.wait()

        def edge(e, c2):
            for j in range(NUM_SCALAR // _L):
                sl = pl.ds(j * _L, _L)
                x_v[e, sl] = x_v[e, sl] * s_v[e, sl] + h_v[e, sl]
            for j in range(NUM_SCALAR // _L, IRREPS_DIM // _L):
                sl = pl.ds(j * _L, _L)
                x_v[e, sl] = x_v[e, sl] * s_v[e, sl]
            return c2

        lax.fori_loop(0, _B, edge, 0, unroll=False)
        pltpu.sync_copy(x_v, out_hbm.at[pl.ds(base, _B)])
        return carry

    lax.fori_loop(0, _CHUNKS, chunk, 0, unroll=False)


@functools.partial(jax.jit, static_argnames=())
def _run(edge_features, etype_flat, scales_exp, shifts):
    mesh = plsc.VectorSubcoreMesh(core_axis_name="c", subcore_axis_name="s")
    call = pl.kernel(
        _sc_body,
        mesh=mesh,
        out_type=jax.ShapeDtypeStruct((N_EDGES, IRREPS_DIM), jnp.float32),
        scratch_types=[
            pltpu.VMEM((_B,), jnp.int32),
            pltpu.VMEM((_B, IRREPS_DIM), jnp.float32),
            pltpu.VMEM((_B, IRREPS_DIM), jnp.float32),
            pltpu.VMEM((_B, IRREPS_DIM), jnp.float32),
            pltpu.SemaphoreType.DMA,
            pltpu.SemaphoreType.DMA,
            pltpu.SemaphoreType.DMA,
        ],
    )
    return call(edge_features, etype_flat, scales_exp, shifts)


def kernel(edge_features, edge_index, edge_type, scales, shifts):
    scales_exp = scales[:, _SCALE_INDEX]
    shifts_pad = jnp.pad(shifts, ((0, 0), (0, IRREPS_DIM - NUM_SCALAR)))
    etype_flat = edge_type.reshape(-1)
    return _run(edge_features, etype_flat, scales_exp, shifts_pad)


# speedup vs baseline: 11.6811x; 1.2541x over previous
import functools

import numpy as np
import jax
import jax.numpy as jnp
from jax import lax
from jax.experimental import pallas as pl
from jax.experimental.pallas import tpu as pltpu
from jax.experimental.pallas import tpu_sc as plsc

NUM_PAIRS = 4096
IRREPS_DIM = 128
NUM_IRREPS = 64
NUM_SCALAR = 32
N_EDGES = 320000

_SHIFT_OFF = NUM_IRREPS

_INFO = plsc.get_sparse_core_info()
_NC, _NS, _L = _INFO.num_cores, _INFO.num_subcores, _INFO.num_lanes
_NW = _NC * _NS
_PER_W = N_EDGES // _NW
_B = 200
_CHUNKS = _PER_W // _B
_NVREG = IRREPS_DIM // _L


def _sc_body(feat_hbm, etype_hbm, table_hbm, out_hbm,
             idx_v, x_v, t_v, sem_x, sem_t):
    wid = lax.axis_index("s") * _NC + lax.axis_index("c")
    w_base = wid * _PER_W

    lane = lax.iota(jnp.int32, _L)
    lidx = []
    for j in range(2, _NVREG):
        col = ((16 * j - 32 + lane) * 10923) >> 15
        lidx.append(col - (0 if j <= 4 else 16))
    dnums = lax.GatherDimensionNumbers(
        offset_dims=(), collapsed_slice_dims=(0,), start_index_map=(0,))

    def perm(v, i):
        return lax.gather(v, i[:, None], dnums, (1,),
                          mode=lax.GatherScatterMode.PROMISE_IN_BOUNDS)

    def chunk(g, carry):
        base = w_base + g * _B
        pltpu.sync_copy(etype_hbm.at[pl.ds(base, _B)], idx_v)
        cp_x = pltpu.async_copy(feat_hbm.at[pl.ds(base, _B)], x_v, sem_x)
        cp_t = pltpu.async_copy(table_hbm.at[idx_v], t_v, sem_t)
        cp_x.wait()
        cp_t.wait()

        def edge(e, c2):
            for j in range(2):
                sl = pl.ds(j * _L, _L)
                hsl = pl.ds(_SHIFT_OFF + j * _L, _L)
                x_v[e, sl] = x_v[e, sl] * t_v[e, sl] + t_v[e, hsl]
            t2 = t_v[e, pl.ds(2 * _L, _L)]
            t3 = t_v[e, pl.ds(3 * _L, _L)]
            for j in range(2, _NVREG):
                sl = pl.ds(j * _L, _L)
                s = perm(t2 if j <= 4 else t3, lidx[j - 2])
                x_v[e, sl] = x_v[e, sl] * s
            return c2

        lax.fori_loop(0, _B, edge, 0, unroll=False)
        pltpu.sync_copy(x_v, out_hbm.at[pl.ds(base, _B)])
        return carry

    lax.fori_loop(0, _CHUNKS, chunk, 0, unroll=False)


@functools.partial(jax.jit, static_argnames=())
def _run(edge_features, etype_flat, table_c):
    mesh = plsc.VectorSubcoreMesh(core_axis_name="c", subcore_axis_name="s")
    call = pl.kernel(
        _sc_body,
        mesh=mesh,
        out_type=jax.ShapeDtypeStruct((N_EDGES, IRREPS_DIM), jnp.float32),
        scratch_types=[
            pltpu.VMEM((_B,), jnp.int32),
            pltpu.VMEM((_B, IRREPS_DIM), jnp.float32),
            pltpu.VMEM((_B, IRREPS_DIM), jnp.float32),
            pltpu.SemaphoreType.DMA,
            pltpu.SemaphoreType.DMA,
        ],
    )
    return call(edge_features, etype_flat, table_c)


def kernel(edge_features, edge_index, edge_type, scales, shifts):
    table_c = jnp.concatenate(
        [scales, shifts, jnp.zeros((NUM_PAIRS, NUM_SCALAR), jnp.float32)],
        axis=1,
    )
    etype_flat = edge_type.reshape(-1)
    return _run(edge_features, etype_flat, table_c)


# speedup vs baseline: 13.9338x; 1.1929x over previous
import functools

import numpy as np
import jax
import jax.numpy as jnp
from jax import lax
from jax.experimental import pallas as pl
from jax.experimental.pallas import tpu as pltpu
from jax.experimental.pallas import tpu_sc as plsc

NUM_PAIRS = 4096
IRREPS_DIM = 128
NUM_IRREPS = 64
NUM_SCALAR = 32
N_EDGES = 320000

_SHIFT_OFF = NUM_IRREPS

_INFO = plsc.get_sparse_core_info()
_NC, _NS, _L = _INFO.num_cores, _INFO.num_subcores, _INFO.num_lanes
_NW = _NC * _NS
_PER_W = N_EDGES // _NW
_B = 80
_CHUNKS = _PER_W // _B
_NVREG = IRREPS_DIM // _L


def _sc_body(feat_hbm, etype_hbm, table_hbm, out_hbm,
             idx_v, x_v, t_v, tab_sh,
             sem_tab, sem_idx, sem_t0, sem_t1, sem_x0, sem_x1,
             sem_o0, sem_o1):
    sid = lax.axis_index("s")
    wid = sid * _NC + lax.axis_index("c")
    w_base = wid * _PER_W

    @pl.when(sid == 0)
    def _():
        pltpu.async_copy(table_hbm, tab_sh, sem_tab).wait()

    pltpu.async_copy(etype_hbm.at[pl.ds(w_base, _PER_W)], idx_v,
                     sem_idx).wait()
    plsc.subcore_barrier()

    lane = lax.iota(jnp.int32, _L)
    lidx = []
    for j in range(2, _NVREG):
        col = ((16 * j - 32 + lane) * 10923) >> 15
        lidx.append(col - (0 if j <= 4 else 16))
    dnums = lax.GatherDimensionNumbers(
        offset_dims=(), collapsed_slice_dims=(0,), start_index_map=(0,))

    def perm(v, i):
        return lax.gather(v, i[:, None], dnums, (1,),
                          mode=lax.GatherScatterMode.PROMISE_IN_BOUNDS)

    sems_t = (sem_t0, sem_t1)
    sems_x = (sem_x0, sem_x1)
    sems_o = (sem_o0, sem_o1)

    def _in_args(g, slot):
        base = w_base + g * _B
        return ((tab_sh.at[idx_v.at[pl.ds(g * _B, _B)]], t_v.at[slot],
                 sems_t[slot]),
                (feat_hbm.at[pl.ds(base, _B)], x_v.at[slot], sems_x[slot]))

    def issue_in(g, slot):
        for a in _in_args(g, slot):
            pltpu.async_copy(*a)

    def wait_in(g, slot):
        for a in _in_args(g, slot):
            pltpu.make_async_copy(*a).wait()

    def _out_args(g, slot):
        base = w_base + g * _B
        return (x_v.at[slot], out_hbm.at[pl.ds(base, _B)], sems_o[slot])

    def issue_out(g, slot):
        pltpu.async_copy(*_out_args(g, slot))

    def wait_out(g, slot):
        pltpu.make_async_copy(*_out_args(g, slot)).wait()

    def compute(slot):
        def edge(e, c2):
            for j in range(2):
                sl = pl.ds(j * _L, _L)
                hsl = pl.ds(_SHIFT_OFF + j * _L, _L)
                x_v[slot, e, sl] = (x_v[slot, e, sl] * t_v[slot, e, sl]
                                    + t_v[slot, e, hsl])
            t2 = t_v[slot, e, pl.ds(2 * _L, _L)]
            t3 = t_v[slot, e, pl.ds(3 * _L, _L)]
            for j in range(2, _NVREG):
                sl = pl.ds(j * _L, _L)
                s = perm(t2 if j <= 4 else t3, lidx[j - 2])
                x_v[slot, e, sl] = x_v[slot, e, sl] * s
            return c2

        lax.fori_loop(0, _B, edge, 0, unroll=False)

    def step(g, slot):
        wait_in(g, slot)
        compute(slot)
        issue_out(g, slot)
        wait_out(g - 1, 1 - slot)
        issue_in(g + 1, 1 - slot)

    issue_in(0, 0)
    issue_in(1, 1)

    wait_in(0, 0)
    compute(0)
    issue_out(0, 0)

    def main(gg, carry):
        for b in (0, 1):
            step(2 * gg + 1 + b, 1 - b)
        return carry

    _M = (_CHUNKS - 3) // 2
    lax.fori_loop(0, _M, main, 0, unroll=False)

    for g in range(2 * _M + 1, _CHUNKS - 1):
        step(g, g % 2)
    g = _CHUNKS - 1
    wait_in(g, g % 2)
    compute(g % 2)
    issue_out(g, g % 2)
    wait_out(g - 1, 1 - g % 2)
    wait_out(g, g % 2)


@functools.partial(jax.jit, static_argnames=())
def _run(edge_features, etype_flat, table_c):
    mesh = plsc.VectorSubcoreMesh(core_axis_name="c", subcore_axis_name="s")
    call = pl.kernel(
        _sc_body,
        mesh=mesh,
        out_type=jax.ShapeDtypeStruct((N_EDGES, IRREPS_DIM), jnp.float32),
        scratch_types=[
            pltpu.VMEM((_PER_W,), jnp.int32),
            pltpu.VMEM((2, _B, IRREPS_DIM), jnp.float32),
            pltpu.VMEM((2, _B, IRREPS_DIM), jnp.float32),
            pltpu.VMEM_SHARED((NUM_PAIRS, IRREPS_DIM), jnp.float32),
            pltpu.SemaphoreType.DMA,
            pltpu.SemaphoreType.DMA,
            pltpu.SemaphoreType.DMA,
            pltpu.SemaphoreType.DMA,
            pltpu.SemaphoreType.DMA,
            pltpu.SemaphoreType.DMA,
            pltpu.SemaphoreType.DMA,
            pltpu.SemaphoreType.DMA,
        ],
    )
    return call(edge_features, etype_flat, table_c)


def kernel(edge_features, edge_index, edge_type, scales, shifts):
    table_c = jnp.concatenate(
        [scales, shifts, jnp.zeros((NUM_PAIRS, NUM_SCALAR), jnp.float32)],
        axis=1,
    )
    etype_flat = edge_type.reshape(-1)
    return _run(edge_features, etype_flat, table_c)
